# Initial kernel scaffold; baseline (speedup 1.0000x reference)
import jax
import jax.numpy as jnp
from jax.experimental import pallas as pl


def kernel(graph, edge_index, x_1, x_2, W1, b1, W2, b2, log_a, log_a_f, b_p, b_f):
    raise NotImplementedError("write your pallas kernel here")


# speedup vs baseline: 12.3677x; 12.3677x over previous
import functools

import jax
import jax.numpy as jnp
from jax import lax
from jax.experimental import pallas as pl
from jax.experimental.pallas import tpu as pltpu
from jax.experimental.pallas import tpu_sc as plsc

N = 10000
E = 320000
NC, NS, LANES = 2, 16, 16
NW = NC * NS
CH = 128
E_PAD = 327680
E_ROWS = E_PAD // CH
CPT32 = E_ROWS // NW
CPT16 = E_ROWS // NS
N_SP = 10112
RPT = N_SP // NS
HPT = 320
H_PAD = HPT * NW

_f32 = jnp.float32
_i32 = jnp.int32


def _mesh():
    return plsc.VectorSubcoreMesh(
        core_axis_name="c", subcore_axis_name="s", num_cores=NC, num_subcores=NS
    )


_SC_PARAMS = pltpu.CompilerParams(use_tc_tiling_on_sc=False)


def _deg_call(dst2d, zeros16, ones16):
    @functools.partial(
        pl.kernel,
        out_type=jax.ShapeDtypeStruct((NC, N_SP, 16), _f32),
        mesh=_mesh(),
        scratch_types=(
            pltpu.VMEM((CPT32, CH), _i32),
            pltpu.VMEM((CH, 16), _f32),
            pltpu.VMEM_SHARED((N_SP, 16), _f32),
            pltpu.SemaphoreType.DMA,
        ),
    )
    def k(dst_h, zer_h, one_h, out_h, dst_v, ones_v, z_sp, ssem):
        c = lax.axis_index("c")
        s = lax.axis_index("s")
        wid = s * NC + c
        row0 = pl.multiple_of(s * RPT, 8)
        pltpu.sync_copy(zer_h.at[pl.ds(row0, RPT)], z_sp.at[pl.ds(row0, RPT)])
        pltpu.sync_copy(dst_h.at[pl.ds(pl.multiple_of(wid * CPT32, 16), CPT32)], dst_v)
        pltpu.sync_copy(one_h, ones_v)
        plsc.subcore_barrier()

        @pl.loop(0, CPT32)
        def _fire(j):
            pltpu.async_copy(ones_v, z_sp.at[dst_v.at[j]], ssem, add=True)

        @pl.loop(0, CPT32)
        def _drain(j):
            pltpu.make_async_copy(ones_v, z_sp.at[dst_v.at[0]], ssem).wait()

        plsc.subcore_barrier()
        pltpu.sync_copy(z_sp.at[pl.ds(row0, RPT)], out_h.at[c, pl.ds(row0, RPT)])

    return k(dst2d, zeros16, ones16)


def _agg_split_call(src2d, dst2d, y2x, zeros64):
    NB = 4

    @functools.partial(
        pl.kernel,
        out_type=jax.ShapeDtypeStruct((NC, N_SP, 64), _f32),
        mesh=_mesh(),
        compiler_params=_SC_PARAMS,
        scratch_types=(
            pltpu.VMEM((CPT16, CH), _i32),
            pltpu.VMEM((CPT16, CH), _i32),
            pltpu.VMEM((NB, CH, 64), _f32),
            pltpu.VMEM_SHARED((N_SP, 64), _f32),
            pltpu.SemaphoreType.DMA((NB,)),
            pltpu.SemaphoreType.DMA((NB,)),
        ),
    )
    def k(src_h, dst_h, y_h, zer_h, out_h, src_v, dst_v, bufs, z_sp, gsem, ssem):
        c = lax.axis_index("c")
        s = lax.axis_index("s")
        row0 = pl.multiple_of(s * RPT, 8)
        pltpu.sync_copy(zer_h.at[pl.ds(row0, RPT)], z_sp.at[pl.ds(row0, RPT)])
        base = pl.multiple_of(s * CPT16, 16)
        pltpu.sync_copy(src_h.at[pl.ds(base, CPT16)], src_v)
        pltpu.sync_copy(dst_h.at[pl.ds(base, CPT16)], dst_v)

        @pl.loop(0, CPT16)
        def _xf(r):
            for u in range(CH // LANES):
                sl = pl.ds(u * LANES, LANES)
                src_v[r, sl] = src_v[r, sl] * 2 + c

        plsc.subcore_barrier()

        for b in range(NB):
            pltpu.async_copy(y_h.at[src_v.at[b]], bufs.at[b], gsem.at[b])

        @pl.loop(0, CPT16, step=NB)
        def _step(i):
            for b in range(NB):
                j = i + b
                pltpu.make_async_copy(y_h.at[src_v.at[0]], bufs.at[b], gsem.at[b]).wait()
                pltpu.async_copy(bufs.at[b], z_sp.at[dst_v.at[j]], ssem.at[b], add=True)
                nxt = j + NB

                @pl.when(nxt < CPT16)
                def _():
                    pltpu.make_async_copy(bufs.at[b], z_sp.at[dst_v.at[0]], ssem.at[b]).wait()
                    pltpu.async_copy(y_h.at[src_v.at[nxt]], bufs.at[b], gsem.at[b])

        for b in range(NB):
            pltpu.make_async_copy(bufs.at[b], z_sp.at[dst_v.at[0]], ssem.at[b]).wait()
        plsc.subcore_barrier()
        pltpu.sync_copy(z_sp.at[pl.ds(row0, RPT)], out_h.at[c, pl.ds(row0, RPT)])

    return k(src2d, dst2d, y2x, zeros64)


def _agg_call(src2d, dst2d, y, zeros64):
    NB = 4

    @functools.partial(
        pl.kernel,
        out_type=jax.ShapeDtypeStruct((NC, N_SP, 64), _f32),
        mesh=_mesh(),
        compiler_params=_SC_PARAMS,
        scratch_types=(
            pltpu.VMEM((CPT32, CH), _i32),
            pltpu.VMEM((CPT32, CH), _i32),
            pltpu.VMEM((NB, CH, 64), _f32),
            pltpu.VMEM_SHARED((N_SP, 64), _f32),
            pltpu.SemaphoreType.DMA((NB,)),
            pltpu.SemaphoreType.DMA((NB,)),
        ),
    )
    def k(src_h, dst_h, y_h, zer_h, out_h, src_v, dst_v, bufs, z_sp, gsem, ssem):
        c = lax.axis_index("c")
        s = lax.axis_index("s")
        wid = s * NC + c
        row0 = pl.multiple_of(s * RPT, 8)
        pltpu.sync_copy(zer_h.at[pl.ds(row0, RPT)], z_sp.at[pl.ds(row0, RPT)])
        base = pl.multiple_of(wid * CPT32, 16)
        pltpu.sync_copy(src_h.at[pl.ds(base, CPT32)], src_v)
        pltpu.sync_copy(dst_h.at[pl.ds(base, CPT32)], dst_v)
        plsc.subcore_barrier()

        for b in range(NB):
            pltpu.async_copy(y_h.at[src_v.at[b]], bufs.at[b], gsem.at[b])

        @pl.loop(0, CPT32, step=NB)
        def _step(i):
            for b in range(NB):
                j = i + b
                pltpu.make_async_copy(y_h.at[src_v.at[0]], bufs.at[b], gsem.at[b]).wait()
                pltpu.async_copy(bufs.at[b], z_sp.at[dst_v.at[j]], ssem.at[b], add=True)
                nxt = j + NB

                @pl.when(nxt < CPT32)
                def _():
                    pltpu.make_async_copy(bufs.at[b], z_sp.at[dst_v.at[0]], ssem.at[b]).wait()
                    pltpu.async_copy(y_h.at[src_v.at[nxt]], bufs.at[b], gsem.at[b])

        for b in range(NB):
            pltpu.make_async_copy(bufs.at[b], z_sp.at[dst_v.at[0]], ssem.at[b]).wait()
        plsc.subcore_barrier()
        pltpu.sync_copy(z_sp.at[pl.ds(row0, RPT)], out_h.at[c, pl.ds(row0, RPT)])

    return k(src2d, dst2d, y, zeros64)


def _head_call(emb, x1p, x2p, scal):
    HCH = HPT // LANES
    d = emb.shape[1]

    @functools.partial(
        pl.kernel,
        out_type=(
            jax.ShapeDtypeStruct((H_PAD,), _f32),
            jax.ShapeDtypeStruct((H_PAD,), _f32),
        ),
        mesh=_mesh(),
        compiler_params=pltpu.CompilerParams(
            use_tc_tiling_on_sc=False, needs_layout_passes=False
        ),
        scratch_types=(
            pltpu.VMEM((HPT,), _i32),
            pltpu.VMEM((HPT,), _i32),
            pltpu.VMEM((2, LANES, 64), _f32),
            pltpu.VMEM((2, LANES, 64), _f32),
            pltpu.VMEM((4, 16), _f32),
            pltpu.VMEM((HPT,), _f32),
            pltpu.VMEM((HPT,), _f32),
            pltpu.SemaphoreType.DMA((2,)),
            pltpu.SemaphoreType.DMA((2,)),
        ),
    )
    def k(emb_h, x1_h, x2_h, sc_h, op_h, of_h,
          x1_v, x2_v, e1_v, e2_v, sc_v, op_v, of_v, sem1, sem2):
        c = lax.axis_index("c")
        s = lax.axis_index("s")
        wid = s * NC + c
        base = pl.multiple_of(wid * HPT, 16)
        pltpu.sync_copy(x1_h.at[pl.ds(base, HPT)], x1_v)
        pltpu.sync_copy(x2_h.at[pl.ds(base, HPT)], x2_v)
        pltpu.sync_copy(sc_h, sc_v)
        a_p = jnp.exp(sc_v[0, :])
        a_f = jnp.exp(sc_v[1, :])
        b_pv = sc_v[2, :]
        b_fv = sc_v[3, :]
        iota = lax.iota(_i32, LANES)

        def fire(kk, b):
            off = pl.multiple_of(kk * LANES, 16)
            pltpu.async_copy(emb_h.at[x1_v.at[pl.ds(off, LANES)]], e1_v.at[b], sem1.at[b])
            pltpu.async_copy(emb_h.at[x2_v.at[pl.ds(off, LANES)]], e2_v.at[b], sem2.at[b])

        for b in range(2):
            fire(b, b)

        @pl.loop(0, HCH, step=2)
        def _step(i):
            for b in range(2):
                kk = i + b
                pltpu.make_async_copy(emb_h.at[x1_v.at[pl.ds(0, LANES)]], e1_v.at[b], sem1.at[b]).wait()
                pltpu.make_async_copy(emb_h.at[x2_v.at[pl.ds(0, LANES)]], e2_v.at[b], sem2.at[b]).wait()
                acc = jnp.zeros((LANES,), _f32)
                for j in range(64):
                    col = jnp.full((LANES,), j, _i32)
                    v1 = plsc.load_gather(e1_v.at[b], [iota, col])
                    v2 = plsc.load_gather(e2_v.at[b], [iota, col])
                    dd = v1 - v2
                    acc = acc + dd * dd
                off = pl.multiple_of(kk * LANES, 16)
                op_v[pl.ds(off, LANES)] = 1.0 / (1.0 + jnp.exp(a_p * acc - b_pv))
                of_v[pl.ds(off, LANES)] = 1.0 / (1.0 + jnp.exp(a_f * acc - b_fv))

                @pl.when(kk + 2 < HCH)
                def _():
                    fire(kk + 2, b)

        pltpu.sync_copy(op_v, op_h.at[pl.ds(base, HPT)])
        pltpu.sync_copy(of_v, of_h.at[pl.ds(base, HPT)])

    return k(emb, x1p, x2p, scal)


_BLK = 1000
_GRID = N // _BLK


def _scale_mm_call(degp, graph, W1):
    def body(dp_ref, x_ref, w_ref, y_ref, dinv_ref):
        deg = dp_ref[0] + dp_ref[1]
        dinv = lax.rsqrt(deg[:, 0:1] + 1.0)
        dinv_ref[...] = dinv
        xw = jnp.dot(x_ref[...], w_ref[...], preferred_element_type=_f32)
        y_ref[...] = dinv * xw

    return pl.pallas_call(
        body,
        grid=(_GRID,),
        in_specs=[
            pl.BlockSpec((2, _BLK, 16), lambda i: (0, i, 0)),
            pl.BlockSpec((_BLK, 128), lambda i: (i, 0)),
            pl.BlockSpec((128, 128), lambda i: (0, 0)),
        ],
        out_specs=[
            pl.BlockSpec((_BLK, 128), lambda i: (i, 0)),
            pl.BlockSpec((_BLK, 1), lambda i: (i, 0)),
        ],
        out_shape=[
            jax.ShapeDtypeStruct((N, 128), _f32),
            jax.ShapeDtypeStruct((N, 1), _f32),
        ],
    )(degp, graph, W1)


def _layer2_call(z1p, y1, dinv, b1r, W2):
    def body(za_ref, zb_ref, y1_ref, dv_ref, b_ref, w_ref, y2_ref):
        dv = dv_ref[...]
        z1 = jnp.concatenate([za_ref[0], zb_ref[0]], axis=1)
        h = dv * (z1 + y1_ref[...]) + b_ref[...]
        h = jnp.maximum(h, 0.0)
        y2_ref[...] = dv * jnp.dot(h, w_ref[...], preferred_element_type=_f32)

    return pl.pallas_call(
        body,
        grid=(_GRID,),
        in_specs=[
            pl.BlockSpec((1, _BLK, 64), lambda i: (0, i, 0)),
            pl.BlockSpec((1, _BLK, 64), lambda i: (1, i, 0)),
            pl.BlockSpec((_BLK, 128), lambda i: (i, 0)),
            pl.BlockSpec((_BLK, 1), lambda i: (i, 0)),
            pl.BlockSpec((1, 128), lambda i: (0, 0)),
            pl.BlockSpec((128, 64), lambda i: (0, 0)),
        ],
        out_specs=pl.BlockSpec((_BLK, 64), lambda i: (i, 0)),
        out_shape=jax.ShapeDtypeStruct((N, 64), _f32),
    )(z1p, z1p, y1, dinv, b1r, W2)


def _emb_call(z2p, y2, dinv, b2r):
    def body(za_ref, zb_ref, y2_ref, dv_ref, b_ref, e_ref):
        e_ref[...] = dv_ref[...] * (za_ref[0] + zb_ref[0] + y2_ref[...]) + b_ref[...]

    return pl.pallas_call(
        body,
        grid=(_GRID,),
        in_specs=[
            pl.BlockSpec((1, _BLK, 64), lambda i: (0, i, 0)),
            pl.BlockSpec((1, _BLK, 64), lambda i: (1, i, 0)),
            pl.BlockSpec((_BLK, 64), lambda i: (i, 0)),
            pl.BlockSpec((_BLK, 1), lambda i: (i, 0)),
            pl.BlockSpec((1, 64), lambda i: (0, 0)),
        ],
        out_specs=pl.BlockSpec((_BLK, 64), lambda i: (i, 0)),
        out_shape=jax.ShapeDtypeStruct((N, 64), _f32),
    )(z2p, z2p, y2, dinv, b2r)


def kernel(graph, edge_index, x_1, x_2, W1, b1, W2, b2, log_a, log_a_f, b_p, b_f):
    src = edge_index[0].astype(_i32)
    dst = edge_index[1].astype(_i32)
    pad = E_PAD - E
    src2d = jnp.concatenate([src, jnp.zeros((pad,), _i32)]).reshape(E_ROWS, CH)
    dst2d = jnp.concatenate([dst, jnp.full((pad,), N, _i32)]).reshape(E_ROWS, CH)

    zeros16 = jnp.zeros((N_SP, 16), _f32)
    ones16 = jnp.ones((CH, 16), _f32)
    zeros64 = jnp.zeros((N_SP, 64), _f32)

    degp = _deg_call(dst2d, zeros16, ones16)
    y1, dinv = _scale_mm_call(degp, graph, W1)
    z1p = _agg_split_call(src2d, dst2d, y1.reshape(2 * N, 64), zeros64)
    y2 = _layer2_call(z1p, y1, dinv, b1.reshape(1, 128), W2)
    z2p = _agg_call(src2d, dst2d, y2, zeros64)
    emb = _emb_call(z2p, y2, dinv, b2.reshape(1, 64))

    hpad = H_PAD - N
    x1p = jnp.concatenate([x_1.astype(_i32), jnp.zeros((hpad,), _i32)])
    x2p = jnp.concatenate([x_2.astype(_i32), jnp.zeros((hpad,), _i32)])
    scal = jnp.stack([
        jnp.broadcast_to(log_a.astype(_f32), (16,)),
        jnp.broadcast_to(log_a_f.astype(_f32), (16,)),
        jnp.broadcast_to(b_p.astype(_f32), (16,)),
        jnp.broadcast_to(b_f.astype(_f32), (16,)),
    ])
    pp, pf = _head_call(emb, x1p, x2p, scal)
    return (pp[:N], pf[:N])


# speedup vs baseline: 13.4655x; 1.0888x over previous
import functools

import jax
import jax.numpy as jnp
from jax import lax
from jax.experimental import pallas as pl
from jax.experimental.pallas import tpu as pltpu
from jax.experimental.pallas import tpu_sc as plsc

N = 10000
E = 320000
NC, NS, LANES = 2, 16, 16
NW = NC * NS
CH = 128
E_PAD = 327680
E_ROWS = E_PAD // CH
CPT32 = E_ROWS // NW
CPT16 = E_ROWS // NS
N_SP = 10112
RPT = N_SP // NS
HPT = 320
H_PAD = HPT * NW

_f32 = jnp.float32
_i32 = jnp.int32


def _mesh():
    return plsc.VectorSubcoreMesh(
        core_axis_name="c", subcore_axis_name="s", num_cores=NC, num_subcores=NS
    )


_SC_PARAMS = pltpu.CompilerParams(use_tc_tiling_on_sc=False)


def _deg_call(dst2d, zeros16, ones16):
    @functools.partial(
        pl.kernel,
        out_type=jax.ShapeDtypeStruct((NC, N_SP, 16), _f32),
        mesh=_mesh(),
        compiler_params=_SC_PARAMS,
        scratch_types=(
            pltpu.VMEM((CPT32, CH), _i32),
            pltpu.VMEM((CH, 16), _f32),
            pltpu.VMEM_SHARED((N_SP, 16), _f32),
            pltpu.SemaphoreType.DMA,
        ),
    )
    def k(dst_h, zer_h, one_h, out_h, dst_v, ones_v, z_sp, ssem):
        c = lax.axis_index("c")
        s = lax.axis_index("s")
        wid = s * NC + c
        row0 = pl.multiple_of(s * RPT, 8)
        pltpu.sync_copy(zer_h.at[pl.ds(row0, RPT)], z_sp.at[pl.ds(row0, RPT)])
        pltpu.sync_copy(dst_h.at[pl.ds(pl.multiple_of(wid * CPT32, 16), CPT32)], dst_v)
        pltpu.sync_copy(one_h, ones_v)
        plsc.subcore_barrier()

        @pl.loop(0, CPT32)
        def _fire(j):
            pltpu.async_copy(ones_v, z_sp.at[dst_v.at[j]], ssem, add=True)

        @pl.loop(0, CPT32)
        def _drain(j):
            pltpu.make_async_copy(ones_v, z_sp.at[dst_v.at[0]], ssem).wait()

        plsc.subcore_barrier()
        pltpu.sync_copy(z_sp.at[pl.ds(row0, RPT)], out_h.at[c, pl.ds(row0, RPT)])

    return k(dst2d, zeros16, ones16)


def _agg_split_call(src2d, dst2d, y2x, zeros64):
    NB = 4

    @functools.partial(
        pl.kernel,
        out_type=jax.ShapeDtypeStruct((NC, N_SP, 64), _f32),
        mesh=_mesh(),
        compiler_params=_SC_PARAMS,
        scratch_types=(
            pltpu.VMEM((CPT16, CH), _i32),
            pltpu.VMEM((CPT16, CH), _i32),
            pltpu.VMEM((NB, CH, 64), _f32),
            pltpu.VMEM_SHARED((N_SP, 64), _f32),
            pltpu.SemaphoreType.DMA((NB,)),
            pltpu.SemaphoreType.DMA((NB,)),
        ),
    )
    def k(src_h, dst_h, y_h, zer_h, out_h, src_v, dst_v, bufs, z_sp, gsem, ssem):
        c = lax.axis_index("c")
        s = lax.axis_index("s")
        row0 = pl.multiple_of(s * RPT, 8)
        pltpu.sync_copy(zer_h.at[pl.ds(row0, RPT)], z_sp.at[pl.ds(row0, RPT)])
        base = pl.multiple_of(s * CPT16, 16)
        pltpu.sync_copy(src_h.at[pl.ds(base, CPT16)], src_v)
        pltpu.sync_copy(dst_h.at[pl.ds(base, CPT16)], dst_v)

        @pl.loop(0, CPT16)
        def _xf(r):
            for u in range(CH // LANES):
                sl = pl.ds(u * LANES, LANES)
                src_v[r, sl] = src_v[r, sl] * 2 + c

        plsc.subcore_barrier()

        for b in range(NB):
            pltpu.async_copy(y_h.at[src_v.at[b]], bufs.at[b], gsem.at[b])

        @pl.loop(0, CPT16, step=NB)
        def _step(i):
            for b in range(NB):
                j = i + b
                pltpu.make_async_copy(y_h.at[src_v.at[0]], bufs.at[b], gsem.at[b]).wait()
                pltpu.async_copy(bufs.at[b], z_sp.at[dst_v.at[j]], ssem.at[b], add=True)
                nxt = j + NB

                @pl.when(nxt < CPT16)
                def _():
                    pltpu.make_async_copy(bufs.at[b], z_sp.at[dst_v.at[0]], ssem.at[b]).wait()
                    pltpu.async_copy(y_h.at[src_v.at[nxt]], bufs.at[b], gsem.at[b])

        for b in range(NB):
            pltpu.make_async_copy(bufs.at[b], z_sp.at[dst_v.at[0]], ssem.at[b]).wait()
        plsc.subcore_barrier()
        pltpu.sync_copy(z_sp.at[pl.ds(row0, RPT)], out_h.at[c, pl.ds(row0, RPT)])

    return k(src2d, dst2d, y2x, zeros64)


def _agg_call(src2d, dst2d, y, zeros64):
    NB = 4

    @functools.partial(
        pl.kernel,
        out_type=jax.ShapeDtypeStruct((NC, N_SP, 64), _f32),
        mesh=_mesh(),
        compiler_params=_SC_PARAMS,
        scratch_types=(
            pltpu.VMEM((CPT32, CH), _i32),
            pltpu.VMEM((CPT32, CH), _i32),
            pltpu.VMEM((NB, CH, 64), _f32),
            pltpu.VMEM_SHARED((N_SP, 64), _f32),
            pltpu.SemaphoreType.DMA((NB,)),
            pltpu.SemaphoreType.DMA((NB,)),
        ),
    )
    def k(src_h, dst_h, y_h, zer_h, out_h, src_v, dst_v, bufs, z_sp, gsem, ssem):
        c = lax.axis_index("c")
        s = lax.axis_index("s")
        wid = s * NC + c
        row0 = pl.multiple_of(s * RPT, 8)
        pltpu.sync_copy(zer_h.at[pl.ds(row0, RPT)], z_sp.at[pl.ds(row0, RPT)])
        base = pl.multiple_of(wid * CPT32, 16)
        pltpu.sync_copy(src_h.at[pl.ds(base, CPT32)], src_v)
        pltpu.sync_copy(dst_h.at[pl.ds(base, CPT32)], dst_v)
        plsc.subcore_barrier()

        for b in range(NB):
            pltpu.async_copy(y_h.at[src_v.at[b]], bufs.at[b], gsem.at[b])

        @pl.loop(0, CPT32, step=NB)
        def _step(i):
            for b in range(NB):
                j = i + b
                pltpu.make_async_copy(y_h.at[src_v.at[0]], bufs.at[b], gsem.at[b]).wait()
                pltpu.async_copy(bufs.at[b], z_sp.at[dst_v.at[j]], ssem.at[b], add=True)
                nxt = j + NB

                @pl.when(nxt < CPT32)
                def _():
                    pltpu.make_async_copy(bufs.at[b], z_sp.at[dst_v.at[0]], ssem.at[b]).wait()
                    pltpu.async_copy(y_h.at[src_v.at[nxt]], bufs.at[b], gsem.at[b])

        for b in range(NB):
            pltpu.make_async_copy(bufs.at[b], z_sp.at[dst_v.at[0]], ssem.at[b]).wait()
        plsc.subcore_barrier()
        pltpu.sync_copy(z_sp.at[pl.ds(row0, RPT)], out_h.at[c, pl.ds(row0, RPT)])

    return k(src2d, dst2d, y, zeros64)


def _head_call(emb, x1p, x2p, scal):
    HCH = HPT // LANES
    d = emb.shape[1]

    @functools.partial(
        pl.kernel,
        out_type=(
            jax.ShapeDtypeStruct((H_PAD,), _f32),
            jax.ShapeDtypeStruct((H_PAD,), _f32),
        ),
        mesh=_mesh(),
        compiler_params=pltpu.CompilerParams(
            use_tc_tiling_on_sc=False, needs_layout_passes=False
        ),
        scratch_types=(
            pltpu.VMEM((HPT,), _i32),
            pltpu.VMEM((HPT,), _i32),
            pltpu.VMEM((2, LANES, 64), _f32),
            pltpu.VMEM((2, LANES, 64), _f32),
            pltpu.VMEM((4, 16), _f32),
            pltpu.VMEM((HPT,), _f32),
            pltpu.VMEM((HPT,), _f32),
            pltpu.SemaphoreType.DMA((2,)),
            pltpu.SemaphoreType.DMA((2,)),
        ),
    )
    def k(emb_h, x1_h, x2_h, sc_h, op_h, of_h,
          x1_v, x2_v, e1_v, e2_v, sc_v, op_v, of_v, sem1, sem2):
        c = lax.axis_index("c")
        s = lax.axis_index("s")
        wid = s * NC + c
        base = pl.multiple_of(wid * HPT, 16)
        pltpu.sync_copy(x1_h.at[pl.ds(base, HPT)], x1_v)
        pltpu.sync_copy(x2_h.at[pl.ds(base, HPT)], x2_v)
        pltpu.sync_copy(sc_h, sc_v)
        a_p = jnp.exp(sc_v[0, :])
        a_f = jnp.exp(sc_v[1, :])
        b_pv = sc_v[2, :]
        b_fv = sc_v[3, :]
        iota = lax.iota(_i32, LANES)

        def fire(kk, b):
            off = pl.multiple_of(kk * LANES, 16)
            pltpu.async_copy(emb_h.at[x1_v.at[pl.ds(off, LANES)]], e1_v.at[b], sem1.at[b])
            pltpu.async_copy(emb_h.at[x2_v.at[pl.ds(off, LANES)]], e2_v.at[b], sem2.at[b])

        for b in range(2):
            fire(b, b)

        @pl.loop(0, HCH, step=2)
        def _step(i):
            for b in range(2):
                kk = i + b
                pltpu.make_async_copy(emb_h.at[x1_v.at[pl.ds(0, LANES)]], e1_v.at[b], sem1.at[b]).wait()
                pltpu.make_async_copy(emb_h.at[x2_v.at[pl.ds(0, LANES)]], e2_v.at[b], sem2.at[b]).wait()
                acc = jnp.zeros((LANES,), _f32)
                for j in range(64):
                    col = jnp.full((LANES,), j, _i32)
                    v1 = plsc.load_gather(e1_v.at[b], [iota, col])
                    v2 = plsc.load_gather(e2_v.at[b], [iota, col])
                    dd = v1 - v2
                    acc = acc + dd * dd
                off = pl.multiple_of(kk * LANES, 16)
                op_v[pl.ds(off, LANES)] = 1.0 / (1.0 + jnp.exp(a_p * acc - b_pv))
                of_v[pl.ds(off, LANES)] = 1.0 / (1.0 + jnp.exp(a_f * acc - b_fv))

                @pl.when(kk + 2 < HCH)
                def _():
                    fire(kk + 2, b)

        pltpu.sync_copy(op_v, op_h.at[pl.ds(base, HPT)])
        pltpu.sync_copy(of_v, of_h.at[pl.ds(base, HPT)])

    return k(emb, x1p, x2p, scal)


_BLK = 1000
_GRID = N // _BLK


def _scale_mm_call(degp, graph, W1):
    def body(dp_ref, x_ref, w_ref, y_ref, dinv_ref):
        deg = dp_ref[0] + dp_ref[1]
        dinv = lax.rsqrt(deg[:, 0:1] + 1.0)
        dinv_ref[...] = dinv
        xw = jnp.dot(x_ref[...], w_ref[...], preferred_element_type=_f32)
        y_ref[...] = dinv * xw

    return pl.pallas_call(
        body,
        grid=(_GRID,),
        in_specs=[
            pl.BlockSpec((2, _BLK, 16), lambda i: (0, i, 0)),
            pl.BlockSpec((_BLK, 128), lambda i: (i, 0)),
            pl.BlockSpec((128, 128), lambda i: (0, 0)),
        ],
        out_specs=[
            pl.BlockSpec((_BLK, 128), lambda i: (i, 0)),
            pl.BlockSpec((_BLK, 1), lambda i: (i, 0)),
        ],
        out_shape=[
            jax.ShapeDtypeStruct((N, 128), _f32),
            jax.ShapeDtypeStruct((N, 1), _f32),
        ],
    )(degp, graph, W1)


def _layer2_call(z1p, y1, dinv, b1r, W2):
    def body(za_ref, zb_ref, y1_ref, dv_ref, b_ref, w_ref, y2_ref):
        dv = dv_ref[...]
        z1 = jnp.concatenate([za_ref[0], zb_ref[0]], axis=1)
        h = dv * (z1 + y1_ref[...]) + b_ref[...]
        h = jnp.maximum(h, 0.0)
        y2_ref[...] = dv * jnp.dot(h, w_ref[...], preferred_element_type=_f32)

    return pl.pallas_call(
        body,
        grid=(_GRID,),
        in_specs=[
            pl.BlockSpec((1, _BLK, 64), lambda i: (0, i, 0)),
            pl.BlockSpec((1, _BLK, 64), lambda i: (1, i, 0)),
            pl.BlockSpec((_BLK, 128), lambda i: (i, 0)),
            pl.BlockSpec((_BLK, 1), lambda i: (i, 0)),
            pl.BlockSpec((1, 128), lambda i: (0, 0)),
            pl.BlockSpec((128, 64), lambda i: (0, 0)),
        ],
        out_specs=pl.BlockSpec((_BLK, 64), lambda i: (i, 0)),
        out_shape=jax.ShapeDtypeStruct((N, 64), _f32),
    )(z1p, z1p, y1, dinv, b1r, W2)


def _emb_call(z2p, y2, dinv, b2r):
    def body(za_ref, zb_ref, y2_ref, dv_ref, b_ref, e_ref):
        e_ref[...] = dv_ref[...] * (za_ref[0] + zb_ref[0] + y2_ref[...]) + b_ref[...]

    return pl.pallas_call(
        body,
        grid=(_GRID,),
        in_specs=[
            pl.BlockSpec((1, _BLK, 64), lambda i: (0, i, 0)),
            pl.BlockSpec((1, _BLK, 64), lambda i: (1, i, 0)),
            pl.BlockSpec((_BLK, 64), lambda i: (i, 0)),
            pl.BlockSpec((_BLK, 1), lambda i: (i, 0)),
            pl.BlockSpec((1, 64), lambda i: (0, 0)),
        ],
        out_specs=pl.BlockSpec((_BLK, 64), lambda i: (i, 0)),
        out_shape=jax.ShapeDtypeStruct((N, 64), _f32),
    )(z2p, z2p, y2, dinv, b2r)


def kernel(graph, edge_index, x_1, x_2, W1, b1, W2, b2, log_a, log_a_f, b_p, b_f):
    src = edge_index[0].astype(_i32)
    dst = edge_index[1].astype(_i32)
    pad = E_PAD - E
    src2d = jnp.concatenate([src, jnp.zeros((pad,), _i32)]).reshape(E_ROWS, CH)
    dst2d = jnp.concatenate([dst, jnp.full((pad,), N, _i32)]).reshape(E_ROWS, CH)

    zeros16 = jnp.zeros((N_SP, 16), _f32)
    ones16 = jnp.ones((CH, 16), _f32)
    zeros64 = jnp.zeros((N_SP, 64), _f32)

    degp = _deg_call(dst2d, zeros16, ones16)
    y1, dinv = _scale_mm_call(degp, graph, W1)
    z1p = _agg_split_call(src2d, dst2d, y1.reshape(2 * N, 64), zeros64)
    y2 = _layer2_call(z1p, y1, dinv, b1.reshape(1, 128), W2)
    z2p = _agg_call(src2d, dst2d, y2, zeros64)
    emb = _emb_call(z2p, y2, dinv, b2.reshape(1, 64))

    hpad = H_PAD - N
    x1p = jnp.concatenate([x_1.astype(_i32), jnp.zeros((hpad,), _i32)])
    x2p = jnp.concatenate([x_2.astype(_i32), jnp.zeros((hpad,), _i32)])
    scal = jnp.stack([
        jnp.broadcast_to(log_a.astype(_f32), (16,)),
        jnp.broadcast_to(log_a_f.astype(_f32), (16,)),
        jnp.broadcast_to(b_p.astype(_f32), (16,)),
        jnp.broadcast_to(b_f.astype(_f32), (16,)),
    ])
    pp, pf = _head_call(emb, x1p, x2p, scal)
    return (pp[:N], pf[:N])
